# Initial kernel scaffold; baseline (speedup 1.0000x reference)
import jax
import jax.numpy as jnp
from jax.experimental import pallas as pl


def kernel(x, edge_index, edge_attr, W_gcn, b_gcn, Ws):
    raise NotImplementedError("write your pallas kernel here")


# speedup vs baseline: 22.4738x; 22.4738x over previous
import functools

import jax
import jax.numpy as jnp
from jax.experimental import pallas as pl
from jax.experimental.pallas import tpu as pltpu

G, N, E, D, OUT, DE = 6, 1024, 32768, 128, 128, 16
GI = G * 4

_INTERPRET = False


def _gcn_body(x_ref, w_ref, b_ref, ws_ref, B_ref, deg_ref,
              gs_ref, xs_ref, acc_ref):
    i = pl.program_id(1)
    xw = jnp.dot(x_ref[0], w_ref[0], preferred_element_type=jnp.float32)
    dis = jax.lax.rsqrt(deg_ref[0, 0] + 1.0)
    z = dis[:, None] * xw
    y = dis[:, None] * jnp.dot(B_ref[0], z, preferred_element_type=jnp.float32)
    y = y + (dis * dis)[:, None] * xw + b_ref[0]
    xt = jnp.maximum(y, 0.0)
    contrib = jnp.dot(xt, ws_ref[0], preferred_element_type=jnp.float32)

    @pl.when(i == 0)
    def _():
        acc_ref[...] = contrib

    @pl.when(i != 0)
    def _():
        acc_ref[...] = acc_ref[...] + contrib

    @pl.when(i == 3)
    def _():
        t = acc_ref[...]
        gs_ref[0, 0] = jnp.mean(t, axis=0)
        nrm = jnp.sqrt(jnp.sum(t * t, axis=1, keepdims=True))
        xs_ref[0] = t / jnp.maximum(nrm, 1e-12)


def _run_gcn(x, W_gcn, b_gcn, Ws, B, deg):
    b3 = b_gcn.reshape(4, 1, OUT)
    gs, xs = pl.pallas_call(
        _gcn_body,
        grid=(G, 4),
        in_specs=[
            pl.BlockSpec((1, N, D), lambda g, i: (g, 0, 0)),
            pl.BlockSpec((1, D, OUT), lambda g, i: (i, 0, 0)),
            pl.BlockSpec((1, 1, OUT), lambda g, i: (i, 0, 0)),
            pl.BlockSpec((1, OUT, OUT), lambda g, i: (i, 0, 0)),
            pl.BlockSpec((1, N, N), lambda g, i: (g * 4 + i, 0, 0)),
            pl.BlockSpec((1, 1, N), lambda g, i: (g * 4 + i, 0, 0)),
        ],
        out_specs=[
            pl.BlockSpec((1, 1, OUT), lambda g, i: (g, 0, 0)),
            pl.BlockSpec((1, N, OUT), lambda g, i: (g, 0, 0)),
        ],
        out_shape=[
            jax.ShapeDtypeStruct((G, 1, OUT), jnp.float32),
            jax.ShapeDtypeStruct((G, N, OUT), jnp.float32),
        ],
        scratch_shapes=[pltpu.VMEM((N, OUT), jnp.float32)],
        interpret=_INTERPRET,
    )(x, W_gcn, b3, Ws, B, deg)
    return gs, xs


def _sim_body(a_ref, b_ref, out_ref):
    p = jax.lax.dot_general(a_ref[0], b_ref[0], (((1,), (1,)), ((), ())),
                            preferred_element_type=jnp.float32)
    t = jnp.tanh(p)
    f = jnp.floor((t + 1.0) * 10.5) - 10.0
    col = jnp.sum(f, axis=0)
    out_ref[0, 0] = jnp.sum(col.reshape(8, OUT), axis=0)


def _run_sim(xs):
    part = pl.pallas_call(
        _sim_body,
        grid=(G, G),
        in_specs=[
            pl.BlockSpec((1, N, OUT), lambda i, j: (i, 0, 0)),
            pl.BlockSpec((1, N, OUT), lambda i, j: (j, 0, 0)),
        ],
        out_specs=pl.BlockSpec((1, 1, OUT), lambda i, j: (i * G + j, 0, 0)),
        out_shape=jax.ShapeDtypeStruct((G * G, 1, OUT), jnp.float32),
        interpret=_INTERPRET,
    )(xs, xs)
    sums = jnp.sum(part.reshape(G * G, OUT), axis=-1)
    return (0.1 / (N * N)) * sums.reshape(G, G)


def _build_B_deg(edge_index, edge_attr):
    def one(g):
        src = edge_index[g, 0]
        dst = edge_index[g, 1]
        flat = dst * N + src

        def per_i(i):
            ew = edge_attr[g, :, i + 2]
            B = jnp.zeros((N * N,), jnp.float32).at[flat].add(ew)
            deg = jnp.zeros((N,), jnp.float32).at[dst].add(ew)
            return B, deg

        Bs, degs = jax.vmap(per_i)(jnp.arange(4))
        return Bs, degs

    Bs, degs = jax.vmap(one)(jnp.arange(G))
    return Bs.reshape(GI, N, N), degs.reshape(GI, 1, N)


def kernel(x, edge_index, edge_attr, W_gcn, b_gcn, Ws):
    B, deg = _build_B_deg(edge_index, edge_attr)
    gs, xs = _run_gcn(x, W_gcn, b_gcn, Ws, B, deg)
    g_matrix = gs.reshape(1, G * OUT)
    sim = _run_sim(xs)
    return g_matrix, sim[None]


# speedup vs baseline: 151.7048x; 6.7503x over previous
import functools

import jax
import jax.numpy as jnp
from jax import lax
from jax.experimental import pallas as pl
from jax.experimental.pallas import tpu as pltpu
from jax.experimental.pallas import tpu_sc as plsc

G, N, E, D, OUT, DE = 6, 1024, 32768, 128, 128, 16
GI = G * 4

_INTERPRET = False


def _gcn_body(x_ref, w_ref, b_ref, ws_ref, B_ref, deg_ref,
              gs_ref, xs_ref, acc_ref):
    i = pl.program_id(1)
    xw = jnp.dot(x_ref[0], w_ref[0], preferred_element_type=jnp.float32)
    dis = jax.lax.rsqrt(deg_ref[0, 0] + 1.0)
    z = dis[:, None] * xw
    y = dis[:, None] * jnp.dot(B_ref[0], z, preferred_element_type=jnp.float32)
    y = y + (dis * dis)[:, None] * xw + b_ref[0]
    xt = jnp.maximum(y, 0.0)
    contrib = jnp.dot(xt, ws_ref[0], preferred_element_type=jnp.float32)

    @pl.when(i == 0)
    def _():
        acc_ref[...] = contrib

    @pl.when(i != 0)
    def _():
        acc_ref[...] = acc_ref[...] + contrib

    @pl.when(i == 3)
    def _():
        t = acc_ref[...]
        gs_ref[0, 0] = jnp.mean(t, axis=0)
        nrm = jnp.sqrt(jnp.sum(t * t, axis=1, keepdims=True))
        xs_ref[0] = t / jnp.maximum(nrm, 1e-12)


def _run_gcn(x, W_gcn, b_gcn, Ws, B, deg):
    b3 = b_gcn.reshape(4, 1, OUT)
    gs, xs = pl.pallas_call(
        _gcn_body,
        grid=(G, 4),
        in_specs=[
            pl.BlockSpec((1, N, D), lambda g, i: (g, 0, 0)),
            pl.BlockSpec((1, D, OUT), lambda g, i: (i, 0, 0)),
            pl.BlockSpec((1, 1, OUT), lambda g, i: (i, 0, 0)),
            pl.BlockSpec((1, OUT, OUT), lambda g, i: (i, 0, 0)),
            pl.BlockSpec((1, N, N), lambda g, i: (g * 4 + i, 0, 0)),
            pl.BlockSpec((1, 1, N), lambda g, i: (g * 4 + i, 0, 0)),
        ],
        out_specs=[
            pl.BlockSpec((1, 1, OUT), lambda g, i: (g, 0, 0)),
            pl.BlockSpec((1, N, OUT), lambda g, i: (g, 0, 0)),
        ],
        out_shape=[
            jax.ShapeDtypeStruct((G, 1, OUT), jnp.float32),
            jax.ShapeDtypeStruct((G, N, OUT), jnp.float32),
        ],
        scratch_shapes=[pltpu.VMEM((N, OUT), jnp.float32)],
        interpret=_INTERPRET,
    )(x, W_gcn, b3, Ws, B, deg)
    return gs, xs


def _sim_body(a_ref, b_ref, out_ref):
    p = jax.lax.dot_general(a_ref[0], b_ref[0], (((1,), (1,)), ((), ())),
                            preferred_element_type=jnp.float32)
    t = jnp.tanh(p)
    f = jnp.floor((t + 1.0) * 10.5) - 10.0
    col = jnp.sum(f, axis=0)
    out_ref[0, 0] = jnp.sum(col.reshape(8, OUT), axis=0)


def _run_sim(xs):
    part = pl.pallas_call(
        _sim_body,
        grid=(G, G),
        in_specs=[
            pl.BlockSpec((1, N, OUT), lambda i, j: (i, 0, 0)),
            pl.BlockSpec((1, N, OUT), lambda i, j: (j, 0, 0)),
        ],
        out_specs=pl.BlockSpec((1, 1, OUT), lambda i, j: (i * G + j, 0, 0)),
        out_shape=jax.ShapeDtypeStruct((G * G, 1, OUT), jnp.float32),
        interpret=_INTERPRET,
    )(xs, xs)
    sums = jnp.sum(part.reshape(G * G, OUT), axis=-1)
    return (0.1 / (N * N)) * sums.reshape(G, G)


_NS = 16
_L = 16
_EPT = E // _NS
_CH = 128
_NCH = _EPT // _CH
_STRIPE = (N * N) // _NS
_ZB = 16384


def _sc_build_body(ei_hbm, ew_hbm, B_hbm, deg_hbm,
                   B_sh, deg_sh, zeros_v, dstv, srcv,
                   idx2d, didx2d, val2d, deg_v):
    c = lax.axis_index("c")
    s = lax.axis_index("s")

    def _zinit(k, _):
        zeros_v[pl.ds(k * _L, _L)] = jnp.zeros((_L,), jnp.float32)
        return _
    lax.fori_loop(0, _ZB // _L, _zinit, None)

    def _round(r, _):
        j = 2 * r + c
        g = j // 4
        i = j % 4

        for q in range(_STRIPE // _ZB):
            pltpu.sync_copy(zeros_v,
                            B_sh.at[pl.ds(s * _STRIPE + q * _ZB, _ZB)])
        pltpu.sync_copy(zeros_v.at[pl.ds(0, N // _NS)],
                        deg_sh.at[pl.ds(s * (N // _NS), N // _NS)])
        plsc.subcore_barrier()

        pltpu.sync_copy(ei_hbm.at[g, 0, pl.ds(s * _EPT, _EPT)], srcv)
        pltpu.sync_copy(ei_hbm.at[g, 1, pl.ds(s * _EPT, _EPT)], dstv)
        pltpu.sync_copy(ew_hbm.at[g, i, s], val2d)

        for t in range(_NCH):
            def _grp(kk, _):
                k = t * (_CH // _L) + kk
                d16 = dstv[pl.ds(k * _L, _L)]
                s16 = srcv[pl.ds(k * _L, _L)]
                idx2d[t, pl.ds(kk * _L, _L)] = d16 * N + s16
                didx2d[t, pl.ds(kk * _L, _L)] = d16
                return _
            lax.fori_loop(0, _CH // _L, _grp, None)

        for t in range(_NCH):
            pltpu.sync_copy(val2d.at[t], B_sh.at[idx2d.at[t]], add=True)
            pltpu.sync_copy(val2d.at[t], deg_sh.at[didx2d.at[t]], add=True)
        plsc.subcore_barrier()

        pltpu.sync_copy(B_sh.at[pl.ds(s * _STRIPE, _STRIPE)],
                        B_hbm.at[j, pl.ds(s * _STRIPE, _STRIPE)])
        pltpu.sync_copy(deg_sh.at[pl.ds(s * (N // _NS), N // _NS)], deg_v)
        pltpu.sync_copy(deg_v,
                        deg_hbm.at[j, pl.ds(s * (N // _NS), N // _NS)])
        return _

    lax.fori_loop(0, GI // 2, _round, None)


def _build_B_deg_sc(edge_index, edge_attr):
    run = pl.kernel(
        _sc_build_body,
        mesh=plsc.VectorSubcoreMesh(core_axis_name="c", subcore_axis_name="s"),
        out_type=[
            jax.ShapeDtypeStruct((GI, N * N), jnp.float32),
            jax.ShapeDtypeStruct((GI, N), jnp.float32),
        ],
        scratch_types=[
            pltpu.VMEM_SHARED((N * N,), jnp.float32),
            pltpu.VMEM_SHARED((N,), jnp.float32),
            pltpu.VMEM((_ZB,), jnp.float32),
            pltpu.VMEM((_EPT,), jnp.int32),
            pltpu.VMEM((_EPT,), jnp.int32),
            pltpu.VMEM((_NCH, _CH), jnp.int32),
            pltpu.VMEM((_NCH, _CH), jnp.int32),
            pltpu.VMEM((_NCH, _CH), jnp.float32),
            pltpu.VMEM((N // _NS,), jnp.float32),
        ],
    )
    ew = jnp.transpose(edge_attr[:, :, 2:6], (0, 2, 1))
    ew = ew.reshape(G, 4, _NS, _NCH, _CH)
    B, deg = run(edge_index, ew)
    return B.reshape(GI, N, N), deg.reshape(GI, 1, N)


def _build_B_deg(edge_index, edge_attr):
    def one(g):
        src = edge_index[g, 0]
        dst = edge_index[g, 1]
        flat = dst * N + src

        def per_i(i):
            ew = edge_attr[g, :, i + 2]
            B = jnp.zeros((N * N,), jnp.float32).at[flat].add(ew)
            deg = jnp.zeros((N,), jnp.float32).at[dst].add(ew)
            return B, deg

        Bs, degs = jax.vmap(per_i)(jnp.arange(4))
        return Bs, degs

    Bs, degs = jax.vmap(one)(jnp.arange(G))
    return Bs.reshape(GI, N, N), degs.reshape(GI, 1, N)


def kernel(x, edge_index, edge_attr, W_gcn, b_gcn, Ws):
    B, deg = _build_B_deg_sc(edge_index, edge_attr)
    gs, xs = _run_gcn(x, W_gcn, b_gcn, Ws, B, deg)
    g_matrix = gs.reshape(1, G * OUT)
    sim = _run_sim(xs)
    return g_matrix, sim[None]
